# Initial kernel scaffold; baseline (speedup 1.0000x reference)
import jax
import jax.numpy as jnp
from jax.experimental import pallas as pl


def kernel(node_features, edge_index, Wq, bq, Wk, bk, Wv, bv, attention, Wo, bo, ln_w, ln_b):
    raise NotImplementedError("write your pallas kernel here")


# speedup vs baseline: 73.8331x; 73.8331x over previous
import functools

import jax
import jax.numpy as jnp
from jax import lax
from jax.experimental import pallas as pl
from jax.experimental.pallas import tpu as pltpu
from jax.experimental.pallas import tpu_sc as plsc

N_NODES = 10000
N_PAD = 10240
N_EDGES = 320000
D = 128
H = 8
DH = 16
ROW = 144
BLK = 1000
PBLK = 1024
N_TC_BLOCKS = N_NODES // BLK
CHUNK = 80
N_WORKERS = 32
EDGES_PER_TILE = N_EDGES // N_WORKERS
CHUNKS_PER_TILE = EDGES_PER_TILE // CHUNK
ROWS_PER_TILE = N_PAD // 16

_HI = lax.Precision.HIGHEST


def _prep_body(x_ref, wvt_ref, bv_ref, aq_w_ref, cq_ref, ak_w_ref, ck_ref,
               vext_ref, aq_ref, ak_ref):
    x = x_ref[...]
    v = lax.dot_general(x, wvt_ref[...], (((1,), (0,)), ((), ())),
                        precision=_HI, preferred_element_type=jnp.float32)
    ones8 = jnp.ones((PBLK, 8), jnp.float32)
    zeros8 = jnp.zeros((PBLK, 8), jnp.float32)
    vext_ref[...] = jnp.concatenate([ones8, zeros8, v + bv_ref[...]], axis=1)
    aq_ref[...] = lax.dot_general(x, aq_w_ref[...], (((1,), (0,)), ((), ())),
                                  precision=_HI,
                                  preferred_element_type=jnp.float32) + cq_ref[...]
    ak_ref[...] = lax.dot_general(x, ak_w_ref[...], (((1,), (0,)), ((), ())),
                                  precision=_HI,
                                  preferred_element_type=jnp.float32) + ck_ref[...]


def _run_prep(x, wvt, bv, aq_w, cq, ak_w, ck):
    full = lambda shape: pl.BlockSpec(shape, lambda i: (0,) * len(shape))
    return pl.pallas_call(
        _prep_body,
        grid=(N_PAD // PBLK,),
        in_specs=[
            pl.BlockSpec((PBLK, D), lambda i: (i, 0)),
            full((D, D)), full((1, D)),
            full((D, DH)), full((1, DH)),
            full((D, DH)), full((1, DH)),
        ],
        out_specs=[
            pl.BlockSpec((PBLK, ROW), lambda i: (i, 0)),
            pl.BlockSpec((PBLK, DH), lambda i: (i, 0)),
            pl.BlockSpec((PBLK, DH), lambda i: (i, 0)),
        ],
        out_shape=[
            jax.ShapeDtypeStruct((N_PAD, ROW), jnp.float32),
            jax.ShapeDtypeStruct((N_PAD, DH), jnp.float32),
            jax.ShapeDtypeStruct((N_PAD, DH), jnp.float32),
        ],
    )(x, wvt, bv, aq_w, cq, ak_w, ck)


def _edge_body(src_hbm, tgt_hbm, aq_hbm, ak_hbm, vext_hbm, zeros_hbm,
               agg_hbm,
               src_idx, tgt_idx, aq_v, ak_v, msg_v, agg_sh,
               sem0, sem1, sem2):
    cid = lax.axis_index("c")
    sid = lax.axis_index("s")
    wid = cid * 16 + sid

    pltpu.sync_copy(zeros_hbm, agg_sh.at[pl.ds(sid * ROWS_PER_TILE,
                                               ROWS_PER_TILE)])

    pltpu.sync_copy(src_hbm.at[wid], src_idx)
    pltpu.sync_copy(tgt_hbm.at[wid], tgt_idx)
    plsc.subcore_barrier()

    def chunk_body(g, carry):
        s_row = src_idx.at[g]
        t_row = tgt_idx.at[g]
        cp0 = pltpu.async_copy(aq_hbm.at[t_row], aq_v, sem0)
        cp1 = pltpu.async_copy(ak_hbm.at[s_row], ak_v, sem1)
        cp2 = pltpu.async_copy(vext_hbm.at[s_row], msg_v, sem2)
        cp0.wait()
        cp1.wait()
        cp2.wait()

        def edge_body(e, c2):
            s16 = aq_v[e, :] + ak_v[e, :]
            s16 = jnp.maximum(s16, 0.2 * s16)
            w16 = jnp.exp(s16)
            msg_v[e, pl.ds(0, 16)] = msg_v[e, pl.ds(0, 16)] * w16
            for h in range(H):
                sl = pl.ds(16 + h * DH, DH)
                msg_v[e, sl] = msg_v[e, sl] * w16[h]
            return c2

        lax.fori_loop(0, CHUNK, edge_body, 0)
        pltpu.sync_copy(msg_v, agg_sh.at[t_row], add=True)
        return carry

    lax.fori_loop(0, CHUNKS_PER_TILE, chunk_body, 0)
    plsc.subcore_barrier()

    pltpu.sync_copy(agg_sh.at[pl.ds(sid * ROWS_PER_TILE, ROWS_PER_TILE)],
                    agg_hbm.at[cid, pl.ds(sid * ROWS_PER_TILE, ROWS_PER_TILE)])


def _run_edges(src2, tgt2, aq, ak, vext, zeros_tile):
    mesh = plsc.VectorSubcoreMesh(core_axis_name="c", subcore_axis_name="s")
    f = functools.partial(
        pl.kernel,
        mesh=mesh,
        compiler_params=pltpu.CompilerParams(use_tc_tiling_on_sc=False),
        out_type=jax.ShapeDtypeStruct((2, N_PAD, ROW), jnp.float32),
        scratch_types=[
            pltpu.VMEM((CHUNKS_PER_TILE, CHUNK), jnp.int32),
            pltpu.VMEM((CHUNKS_PER_TILE, CHUNK), jnp.int32),
            pltpu.VMEM((CHUNK, DH), jnp.float32),
            pltpu.VMEM((CHUNK, DH), jnp.float32),
            pltpu.VMEM((CHUNK, ROW), jnp.float32),
            pltpu.VMEM_SHARED((N_PAD, ROW), jnp.float32),
            pltpu.SemaphoreType.DMA,
            pltpu.SemaphoreType.DMA,
            pltpu.SemaphoreType.DMA,
        ],
    )(_edge_body)
    return f(src2, tgt2, aq, ak, vext, zeros_tile)


def _epi_body(ssum2_ref, aggv2_ref, x_ref, wot_ref, bo_ref, lnw_ref, lnb_ref,
              rexp_ref, out_ref):
    ss = ssum2_ref[0] + ssum2_ref[1]
    ss = jnp.where(ss == 0.0, 1.0, ss)
    rec = 1.0 / ss
    recx = lax.dot_general(rec, rexp_ref[...], (((1,), (0,)), ((), ())),
                           precision=_HI, preferred_element_type=jnp.float32)
    av = (aggv2_ref[0] + aggv2_ref[1]) * recx
    y = lax.dot_general(av, wot_ref[...], (((1,), (0,)), ((), ())),
                        precision=_HI, preferred_element_type=jnp.float32)
    y = y + bo_ref[...] + x_ref[...]
    mu = jnp.mean(y, axis=1, keepdims=True)
    yc = y - mu
    var = jnp.mean(yc * yc, axis=1, keepdims=True)
    out_ref[...] = yc * lax.rsqrt(var + 1e-5) * lnw_ref[...] + lnb_ref[...]


def _run_epilogue(ssum2, aggv2, x, wot, bo, lnw, lnb, rexp):
    full = lambda shape: pl.BlockSpec(shape, lambda i: (0,) * len(shape))
    return pl.pallas_call(
        _epi_body,
        grid=(N_TC_BLOCKS,),
        in_specs=[
            pl.BlockSpec((2, BLK, H), lambda i: (0, i, 0)),
            pl.BlockSpec((2, BLK, D), lambda i: (0, i, 0)),
            pl.BlockSpec((BLK, D), lambda i: (i, 0)),
            full((D, D)), full((1, D)), full((1, D)), full((1, D)),
            full((H, D)),
        ],
        out_specs=pl.BlockSpec((BLK, D), lambda i: (i, 0)),
        out_shape=jax.ShapeDtypeStruct((N_NODES, D), jnp.float32),
    )(ssum2, aggv2, x, wot, bo, lnw, lnb, rexp)


def kernel(node_features, edge_index, Wq, bq, Wk, bk, Wv, bv, attention,
           Wo, bo, ln_w, ln_b):
    x = node_features[0]
    att_q = attention[:, :DH]
    att_k = attention[:, DH:]

    aq_w = jnp.einsum("hd,hdi->ih", att_q, Wq.reshape(H, DH, D))
    ak_w = jnp.einsum("hd,hdi->ih", att_k, Wk.reshape(H, DH, D))
    pad = jnp.zeros((D, H), jnp.float32)
    aq_w16 = jnp.concatenate([aq_w, pad], axis=1)
    ak_w16 = jnp.concatenate([ak_w, pad], axis=1)
    cq = (bq.reshape(H, DH) * att_q).sum(-1)
    ck = (bk.reshape(H, DH) * att_k).sum(-1)
    cq16 = jnp.concatenate([cq, jnp.zeros((H,), jnp.float32)]).reshape(1, 2 * H)
    ck16 = jnp.concatenate([ck, jnp.zeros((H,), jnp.float32)]).reshape(1, 2 * H)

    x_pad = jnp.concatenate(
        [x, jnp.zeros((N_PAD - N_NODES, D), jnp.float32)], axis=0)
    vext, aq, ak = _run_prep(x_pad, Wv.T, bv.reshape(1, D), aq_w16, cq16,
                             ak_w16, ck16)

    src2 = edge_index[0, 0].astype(jnp.int32).reshape(
        N_WORKERS, CHUNKS_PER_TILE, CHUNK)
    tgt2 = edge_index[0, 1].astype(jnp.int32).reshape(
        N_WORKERS, CHUNKS_PER_TILE, CHUNK)
    zeros_tile = jnp.zeros((ROWS_PER_TILE, ROW), jnp.float32)

    agg2 = _run_edges(src2, tgt2, aq, ak, vext, zeros_tile)

    ssum2 = agg2[:, :N_NODES, :H]
    aggv2 = agg2[:, :N_NODES, 16:]
    rexp = jnp.repeat(jnp.eye(H, dtype=jnp.float32), DH, axis=1)
    out = _run_epilogue(ssum2, aggv2, x, Wo.T, bo.reshape(1, D),
                        ln_w.reshape(1, D), ln_b.reshape(1, D), rexp)
    return out.reshape(1, N_NODES, D)


# speedup vs baseline: 94.9687x; 1.2863x over previous
import functools

import jax
import jax.numpy as jnp
from jax import lax
from jax.experimental import pallas as pl
from jax.experimental.pallas import tpu as pltpu
from jax.experimental.pallas import tpu_sc as plsc

N_NODES = 10000
N_PAD = 10240
N_EDGES = 320000
D = 128
H = 8
DH = 16
ROW = 144
BLK = 1000
PBLK = 1024
N_TC_BLOCKS = N_NODES // BLK
CHUNK = 40
N_WORKERS = 32
EDGES_PER_TILE = N_EDGES // N_WORKERS
CHUNKS_PER_TILE = EDGES_PER_TILE // CHUNK
ROWS_PER_TILE = N_PAD // 16

_HI = lax.Precision.HIGHEST


def _prep_body(x_ref, wvt_ref, bv_ref, aq_w_ref, cq_ref, ak_w_ref, ck_ref,
               vext_ref, aq_ref, ak_ref):
    x = x_ref[...]
    v = lax.dot_general(x, wvt_ref[...], (((1,), (0,)), ((), ())),
                        precision=_HI, preferred_element_type=jnp.float32)
    ones8 = jnp.ones((PBLK, 8), jnp.float32)
    zeros8 = jnp.zeros((PBLK, 8), jnp.float32)
    vext_ref[...] = jnp.concatenate([ones8, zeros8, v + bv_ref[...]], axis=1)
    aq_ref[...] = lax.dot_general(x, aq_w_ref[...], (((1,), (0,)), ((), ())),
                                  precision=_HI,
                                  preferred_element_type=jnp.float32) + cq_ref[...]
    ak_ref[...] = lax.dot_general(x, ak_w_ref[...], (((1,), (0,)), ((), ())),
                                  precision=_HI,
                                  preferred_element_type=jnp.float32) + ck_ref[...]


def _run_prep(x, wvt, bv, aq_w, cq, ak_w, ck):
    full = lambda shape: pl.BlockSpec(shape, lambda i: (0,) * len(shape))
    return pl.pallas_call(
        _prep_body,
        grid=(N_PAD // PBLK,),
        in_specs=[
            pl.BlockSpec((PBLK, D), lambda i: (i, 0)),
            full((D, D)), full((1, D)),
            full((D, DH)), full((1, DH)),
            full((D, DH)), full((1, DH)),
        ],
        out_specs=[
            pl.BlockSpec((PBLK, ROW), lambda i: (i, 0)),
            pl.BlockSpec((PBLK, DH), lambda i: (i, 0)),
            pl.BlockSpec((PBLK, DH), lambda i: (i, 0)),
        ],
        out_shape=[
            jax.ShapeDtypeStruct((N_PAD, ROW), jnp.float32),
            jax.ShapeDtypeStruct((N_PAD, DH), jnp.float32),
            jax.ShapeDtypeStruct((N_PAD, DH), jnp.float32),
        ],
    )(x, wvt, bv, aq_w, cq, ak_w, ck)


NBUF = 2


def _edge_body(src_hbm, tgt_hbm, aq_hbm, ak_hbm, vext_hbm, zeros_hbm,
               agg_hbm,
               src_idx, tgt_idx, aq_l, ak_l, msg_l, agg_sh,
               gq_l, gk_l, gv_l, sc_l):
    cid = lax.axis_index("c")
    sid = lax.axis_index("s")
    wid = cid * 16 + sid

    pltpu.sync_copy(zeros_hbm, agg_sh.at[pl.ds(sid * ROWS_PER_TILE,
                                               ROWS_PER_TILE)])

    pltpu.sync_copy(src_hbm.at[wid], src_idx)
    pltpu.sync_copy(tgt_hbm.at[wid], tgt_idx)
    plsc.subcore_barrier()

    def issue(g, b):
        pltpu.async_copy(aq_hbm.at[tgt_idx.at[g]], aq_l[b], gq_l[b])
        pltpu.async_copy(ak_hbm.at[src_idx.at[g]], ak_l[b], gk_l[b])
        pltpu.async_copy(vext_hbm.at[src_idx.at[g]], msg_l[b], gv_l[b])

    def wait_gathers(g, b):
        pltpu.make_async_copy(aq_hbm.at[tgt_idx.at[g]], aq_l[b],
                              gq_l[b]).wait()
        pltpu.make_async_copy(ak_hbm.at[src_idx.at[g]], ak_l[b],
                              gk_l[b]).wait()
        pltpu.make_async_copy(vext_hbm.at[src_idx.at[g]], msg_l[b],
                              gv_l[b]).wait()

    def wait_scatter(g, b):
        pltpu.make_async_copy(msg_l[b], agg_sh.at[tgt_idx.at[g]],
                              sc_l[b]).wait()

    issue(0, 0)

    def pair_body(p, carry):
        for b in range(NBUF):
            g = NBUF * p + b
            bn = (b + 1) % NBUF

            @pl.when(g + 1 < CHUNKS_PER_TILE)
            def _():
                @pl.when(g + 1 >= NBUF)
                def _():
                    wait_scatter(g, bn)
                issue(g + 1, bn)

            wait_gathers(g, b)
            aq_v, ak_v, msg_v = aq_l[b], ak_l[b], msg_l[b]

            def edge_body(e, c2):
                s16 = aq_v[e, :] + ak_v[e, :]
                s16 = jnp.maximum(s16, 0.2 * s16)
                w16 = jnp.exp(s16)
                msg_v[e, pl.ds(0, 16)] = msg_v[e, pl.ds(0, 16)] * w16
                for h in range(H):
                    sl = pl.ds(16 + h * DH, DH)
                    msg_v[e, sl] = msg_v[e, sl] * w16[h]
                return c2

            lax.fori_loop(0, CHUNK, edge_body, 0, unroll=2)
            pltpu.async_copy(msg_v, agg_sh.at[tgt_idx.at[g]], sc_l[b],
                             add=True)
        return carry

    lax.fori_loop(0, CHUNKS_PER_TILE // NBUF, pair_body, 0)
    for b in range(NBUF):
        wait_scatter(CHUNKS_PER_TILE - NBUF + b, b)
    plsc.subcore_barrier()

    pltpu.sync_copy(agg_sh.at[pl.ds(sid * ROWS_PER_TILE, ROWS_PER_TILE)],
                    agg_hbm.at[cid, pl.ds(sid * ROWS_PER_TILE, ROWS_PER_TILE)])


def _run_edges(src2, tgt2, aq, ak, vext, zeros_tile):
    mesh = plsc.VectorSubcoreMesh(core_axis_name="c", subcore_axis_name="s")
    f = functools.partial(
        pl.kernel,
        mesh=mesh,
        compiler_params=pltpu.CompilerParams(use_tc_tiling_on_sc=False),
        out_type=jax.ShapeDtypeStruct((2, N_PAD, ROW), jnp.float32),
        scratch_types=[
            pltpu.VMEM((CHUNKS_PER_TILE, CHUNK), jnp.int32),
            pltpu.VMEM((CHUNKS_PER_TILE, CHUNK), jnp.int32),
            [pltpu.VMEM((CHUNK, DH), jnp.float32) for _ in range(NBUF)],
            [pltpu.VMEM((CHUNK, DH), jnp.float32) for _ in range(NBUF)],
            [pltpu.VMEM((CHUNK, ROW), jnp.float32) for _ in range(NBUF)],
            pltpu.VMEM_SHARED((N_PAD, ROW), jnp.float32),
            [pltpu.SemaphoreType.DMA for _ in range(NBUF)],
            [pltpu.SemaphoreType.DMA for _ in range(NBUF)],
            [pltpu.SemaphoreType.DMA for _ in range(NBUF)],
            [pltpu.SemaphoreType.DMA for _ in range(NBUF)],
        ],
    )(_edge_body)
    return f(src2, tgt2, aq, ak, vext, zeros_tile)


def _epi_body(ssum2_ref, aggv2_ref, x_ref, wot_ref, bo_ref, lnw_ref, lnb_ref,
              rexp_ref, out_ref):
    ss = ssum2_ref[0] + ssum2_ref[1]
    ss = jnp.where(ss == 0.0, 1.0, ss)
    rec = 1.0 / ss
    recx = lax.dot_general(rec, rexp_ref[...], (((1,), (0,)), ((), ())),
                           precision=_HI, preferred_element_type=jnp.float32)
    av = (aggv2_ref[0] + aggv2_ref[1]) * recx
    y = lax.dot_general(av, wot_ref[...], (((1,), (0,)), ((), ())),
                        precision=_HI, preferred_element_type=jnp.float32)
    y = y + bo_ref[...] + x_ref[...]
    mu = jnp.mean(y, axis=1, keepdims=True)
    yc = y - mu
    var = jnp.mean(yc * yc, axis=1, keepdims=True)
    out_ref[...] = yc * lax.rsqrt(var + 1e-5) * lnw_ref[...] + lnb_ref[...]


def _run_epilogue(ssum2, aggv2, x, wot, bo, lnw, lnb, rexp):
    full = lambda shape: pl.BlockSpec(shape, lambda i: (0,) * len(shape))
    return pl.pallas_call(
        _epi_body,
        grid=(N_TC_BLOCKS,),
        in_specs=[
            pl.BlockSpec((2, BLK, H), lambda i: (0, i, 0)),
            pl.BlockSpec((2, BLK, D), lambda i: (0, i, 0)),
            pl.BlockSpec((BLK, D), lambda i: (i, 0)),
            full((D, D)), full((1, D)), full((1, D)), full((1, D)),
            full((H, D)),
        ],
        out_specs=pl.BlockSpec((BLK, D), lambda i: (i, 0)),
        out_shape=jax.ShapeDtypeStruct((N_NODES, D), jnp.float32),
    )(ssum2, aggv2, x, wot, bo, lnw, lnb, rexp)


def kernel(node_features, edge_index, Wq, bq, Wk, bk, Wv, bv, attention,
           Wo, bo, ln_w, ln_b):
    x = node_features[0]
    att_q = attention[:, :DH]
    att_k = attention[:, DH:]

    aq_w = jnp.einsum("hd,hdi->ih", att_q, Wq.reshape(H, DH, D))
    ak_w = jnp.einsum("hd,hdi->ih", att_k, Wk.reshape(H, DH, D))
    pad = jnp.zeros((D, H), jnp.float32)
    aq_w16 = jnp.concatenate([aq_w, pad], axis=1)
    ak_w16 = jnp.concatenate([ak_w, pad], axis=1)
    cq = (bq.reshape(H, DH) * att_q).sum(-1)
    ck = (bk.reshape(H, DH) * att_k).sum(-1)
    cq16 = jnp.concatenate([cq, jnp.zeros((H,), jnp.float32)]).reshape(1, 2 * H)
    ck16 = jnp.concatenate([ck, jnp.zeros((H,), jnp.float32)]).reshape(1, 2 * H)

    x_pad = jnp.concatenate(
        [x, jnp.zeros((N_PAD - N_NODES, D), jnp.float32)], axis=0)
    vext, aq, ak = _run_prep(x_pad, Wv.T, bv.reshape(1, D), aq_w16, cq16,
                             ak_w16, ck16)

    src2 = edge_index[0, 0].astype(jnp.int32).reshape(
        N_WORKERS, CHUNKS_PER_TILE, CHUNK)
    tgt2 = edge_index[0, 1].astype(jnp.int32).reshape(
        N_WORKERS, CHUNKS_PER_TILE, CHUNK)
    zeros_tile = jnp.zeros((ROWS_PER_TILE, ROW), jnp.float32)

    agg2 = _run_edges(src2, tgt2, aq, ak, vext, zeros_tile)

    ssum2 = agg2[:, :N_NODES, :H]
    aggv2 = agg2[:, :N_NODES, 16:]
    rexp = jnp.repeat(jnp.eye(H, dtype=jnp.float32), DH, axis=1)
    out = _run_epilogue(ssum2, aggv2, x, Wo.T, bo.reshape(1, D),
                        ln_w.reshape(1, D), ln_b.reshape(1, D), rexp)
    return out.reshape(1, N_NODES, D)
